# Initial kernel scaffold; baseline (speedup 1.0000x reference)
import jax
import jax.numpy as jnp
from jax.experimental import pallas as pl


def kernel(features_0, features_1, features_2, W0, b0, W1, b1, W2, b2):
    raise NotImplementedError("write your pallas kernel here")


# speedup vs baseline: 1.0744x; 1.0744x over previous
import jax
import jax.numpy as jnp
from jax.experimental import pallas as pl
from jax.experimental.pallas import tpu as pltpu

_NA = 3
_NC = 80
_NO = 5 + _NC
_HWS = [64 * 64, 32 * 32, 16 * 16]
_TOTAL_ROWS = _NA * sum(_HWS)
_ROW_OFF = [0, _NA * _HWS[0], _NA * (_HWS[0] + _HWS[1])]


def _body(feat_ref, w_ref, b_ref, out_ref):
    a = pl.program_id(2)
    x = feat_ref[0]
    w = w_ref[a]
    y = jax.lax.dot_general(
        x, w, (((0,), (1,)), ((), ())), preferred_element_type=jnp.float32
    )
    out_ref[0] = y + b_ref[a][None, :]


def _body_acc(feat_ref, w_ref, b_ref, acc_ref, out_ref):
    del acc_ref
    _body(feat_ref, w_ref, b_ref, out_ref)


def _scale_call(feat, w3, b2, acc, tile, row_off):
    B, C, HW = feat.shape
    T = HW // tile
    grid = (B, T, _NA)

    feat_spec = pl.BlockSpec((1, C, tile), lambda b, t, a: (b, 0, t))
    w_spec = pl.BlockSpec((_NA, _NO, C), lambda b, t, a: (0, 0, 0))
    b_spec = pl.BlockSpec((_NA, _NO), lambda b, t, a: (0, 0))
    out_spec = pl.BlockSpec(
        (1, tile, _NO),
        lambda b, t, a, _o=row_off // tile, _n=HW // tile: (b, _o + a * _n + t, 0),
    )
    out_shape = jax.ShapeDtypeStruct((B, _TOTAL_ROWS, _NO), jnp.float32)

    if acc is None:
        return pl.pallas_call(
            _body,
            grid=grid,
            in_specs=[feat_spec, w_spec, b_spec],
            out_specs=out_spec,
            out_shape=out_shape,
        )(feat, w3, b2)
    return pl.pallas_call(
        _body_acc,
        grid=grid,
        in_specs=[feat_spec, w_spec, b_spec,
                  pl.BlockSpec(memory_space=pl.ANY)],
        out_specs=out_spec,
        out_shape=out_shape,
        input_output_aliases={3: 0},
    )(feat, w3, b2, acc)


def kernel(features_0, features_1, features_2, W0, b0, W1, b1, W2, b2):
    feats = [features_0, features_1, features_2]
    Ws = [W0, W1, W2]
    bs = [b0, b1, b2]
    tiles = [1024, 512, 256]

    out = None
    for i in range(3):
        B, C, H, Wd = feats[i].shape
        feat = feats[i].reshape(B, C, H * Wd)
        w3 = Ws[i].reshape(_NA, _NO, C)
        b2 = bs[i].reshape(_NA, _NO)
        out = _scale_call(feat, w3, b2, out, tiles[i], _ROW_OFF[i])
    return out


# speedup vs baseline: 1.1010x; 1.0248x over previous
import jax
import jax.numpy as jnp
from jax.experimental import pallas as pl
from jax.experimental.pallas import tpu as pltpu

_NA = 3
_NC = 80
_NO = 5 + _NC
_HWS = [64 * 64, 32 * 32, 16 * 16]
_TOTAL_ROWS = _NA * sum(_HWS)
_ROW_OFF = [0, _NA * _HWS[0], _NA * (_HWS[0] + _HWS[1])]


def _body(feat_ref, w_ref, b_ref, out_ref):
    a = pl.program_id(2)
    x = feat_ref[0].astype(jnp.bfloat16)
    w = w_ref[a].astype(jnp.bfloat16)
    y = jax.lax.dot_general(
        x, w, (((0,), (1,)), ((), ())), preferred_element_type=jnp.float32
    )
    out_ref[0] = y + b_ref[a][None, :]


def _body_acc(feat_ref, w_ref, b_ref, acc_ref, out_ref):
    del acc_ref
    _body(feat_ref, w_ref, b_ref, out_ref)


def _scale_call(feat, w3, b2, acc, tile, row_off):
    B, C, HW = feat.shape
    T = HW // tile
    grid = (B, T, _NA)

    feat_spec = pl.BlockSpec((1, C, tile), lambda b, t, a: (b, 0, t))
    w_spec = pl.BlockSpec((_NA, _NO, C), lambda b, t, a: (0, 0, 0))
    b_spec = pl.BlockSpec((_NA, _NO), lambda b, t, a: (0, 0))
    out_spec = pl.BlockSpec(
        (1, tile, _NO),
        lambda b, t, a, _o=row_off // tile, _n=HW // tile: (b, _o + a * _n + t, 0),
    )
    out_shape = jax.ShapeDtypeStruct((B, _TOTAL_ROWS, _NO), jnp.float32)

    if acc is None:
        return pl.pallas_call(
            _body,
            grid=grid,
            in_specs=[feat_spec, w_spec, b_spec],
            out_specs=out_spec,
            out_shape=out_shape,
        )(feat, w3, b2)
    return pl.pallas_call(
        _body_acc,
        grid=grid,
        in_specs=[feat_spec, w_spec, b_spec,
                  pl.BlockSpec(memory_space=pl.ANY)],
        out_specs=out_spec,
        out_shape=out_shape,
        input_output_aliases={3: 0},
    )(feat, w3, b2, acc)


def kernel(features_0, features_1, features_2, W0, b0, W1, b1, W2, b2):
    feats = [features_0, features_1, features_2]
    Ws = [W0, W1, W2]
    bs = [b0, b1, b2]
    tiles = [1024, 512, 256]

    out = None
    for i in range(3):
        B, C, H, Wd = feats[i].shape
        feat = feats[i].reshape(B, C, H * Wd)
        w3 = Ws[i].reshape(_NA, _NO, C)
        b2 = bs[i].reshape(_NA, _NO)
        out = _scale_call(feat, w3, b2, out, tiles[i], _ROW_OFF[i])
    return out


# speedup vs baseline: 1.1016x; 1.0005x over previous
import jax
import jax.numpy as jnp
from jax.experimental import pallas as pl
from jax.experimental.pallas import tpu as pltpu

_NA = 3
_NC = 80
_NO = 5 + _NC
_HWS = [64 * 64, 32 * 32, 16 * 16]
_TOTAL_ROWS = _NA * sum(_HWS)
_ROW_OFF = [0, _NA * _HWS[0], _NA * (_HWS[0] + _HWS[1])]


def _body(feat_ref, w_ref, b_ref, out_ref):
    a = pl.program_id(2)
    x = feat_ref[0].astype(jnp.bfloat16)
    w = w_ref[a].astype(jnp.bfloat16)
    y = jax.lax.dot_general(
        x, w, (((0,), (1,)), ((), ())), preferred_element_type=jnp.float32
    )
    out_ref[0] = y + b_ref[a][None, :]


def _body_acc(feat_ref, w_ref, b_ref, acc_ref, out_ref):
    del acc_ref
    _body(feat_ref, w_ref, b_ref, out_ref)


def _scale_call(feat, w3, b2, acc, tile, row_off):
    B, C, HW = feat.shape
    T = HW // tile
    grid = (B, T, _NA)

    feat_spec = pl.BlockSpec((1, C, tile), lambda b, t, a: (b, 0, t))
    w_spec = pl.BlockSpec((_NA, _NO, C), lambda b, t, a: (0, 0, 0))
    b_spec = pl.BlockSpec((_NA, _NO), lambda b, t, a: (0, 0))
    out_spec = pl.BlockSpec(
        (1, tile, _NO),
        lambda b, t, a, _o=row_off // tile, _n=HW // tile: (b, _o + a * _n + t, 0),
    )
    out_shape = jax.ShapeDtypeStruct((B, _TOTAL_ROWS, _NO), jnp.float32)

    params = pltpu.CompilerParams(
        dimension_semantics=("parallel", "parallel", "arbitrary")
    )
    if acc is None:
        return pl.pallas_call(
            _body,
            grid=grid,
            in_specs=[feat_spec, w_spec, b_spec],
            out_specs=out_spec,
            out_shape=out_shape,
            compiler_params=params,
        )(feat, w3, b2)
    return pl.pallas_call(
        _body_acc,
        grid=grid,
        in_specs=[feat_spec, w_spec, b_spec,
                  pl.BlockSpec(memory_space=pl.ANY)],
        out_specs=out_spec,
        out_shape=out_shape,
        input_output_aliases={3: 0},
        compiler_params=params,
    )(feat, w3, b2, acc)


def kernel(features_0, features_1, features_2, W0, b0, W1, b1, W2, b2):
    feats = [features_0, features_1, features_2]
    Ws = [W0, W1, W2]
    bs = [b0, b1, b2]
    tiles = [1024, 512, 256]

    out = None
    for i in range(3):
        B, C, H, Wd = feats[i].shape
        feat = feats[i].reshape(B, C, H * Wd)
        w3 = Ws[i].reshape(_NA, _NO, C)
        b2 = bs[i].reshape(_NA, _NO)
        out = _scale_call(feat, w3, b2, out, tiles[i], _ROW_OFF[i])
    return out


# speedup vs baseline: 1.3970x; 1.2682x over previous
import jax
import jax.numpy as jnp
from jax.experimental import pallas as pl
from jax.experimental.pallas import tpu as pltpu

_NA = 3
_NC = 80
_NO = 5 + _NC
_HWS = [64 * 64, 32 * 32, 16 * 16]
_TOTAL_ROWS = _NA * sum(_HWS)
_ROW_OFF = [0, _NA * _HWS[0], _NA * (_HWS[0] + _HWS[1])]


def _body(feat_ref, w_ref, b_ref, out_ref):
    a = pl.program_id(2)
    x = feat_ref[0].astype(jnp.bfloat16)
    w = w_ref[a].astype(jnp.bfloat16)
    y = jax.lax.dot_general(
        x, w, (((0,), (1,)), ((), ())), preferred_element_type=jnp.float32
    )
    out_ref[0] = y + b_ref[a][None, :]


def _body_acc(feat_ref, w_ref, b_ref, acc_ref, out_ref):
    del acc_ref
    _body(feat_ref, w_ref, b_ref, out_ref)


def _scale_call(feat, w3, b2, acc, tile, row_off):
    B, C, HW = feat.shape
    T = HW // tile
    grid = (B, T, _NA)

    feat_spec = pl.BlockSpec((1, C, tile), lambda b, t, a: (b, 0, t))
    w_spec = pl.BlockSpec((_NA, _NO, C), lambda b, t, a: (0, 0, 0))
    b_spec = pl.BlockSpec((_NA, _NO), lambda b, t, a: (0, 0))
    out_spec = pl.BlockSpec(
        (1, tile, _NO),
        lambda b, t, a, _o=row_off // tile, _n=HW // tile: (b, _o + a * _n + t, 0),
    )
    out_shape = jax.ShapeDtypeStruct((B, _TOTAL_ROWS, _NO), jnp.float32)

    params = pltpu.CompilerParams(
        dimension_semantics=("parallel", "parallel", "arbitrary")
    )
    if acc is None:
        return pl.pallas_call(
            _body,
            grid=grid,
            in_specs=[feat_spec, w_spec, b_spec],
            out_specs=out_spec,
            out_shape=out_shape,
            compiler_params=params,
        )(feat, w3, b2)
    return pl.pallas_call(
        _body_acc,
        grid=grid,
        in_specs=[feat_spec, w_spec, b_spec,
                  pl.BlockSpec(memory_space=pl.ANY)],
        out_specs=out_spec,
        out_shape=out_shape,
        input_output_aliases={3: 0},
        compiler_params=params,
    )(feat, w3, b2, acc)


def kernel(features_0, features_1, features_2, W0, b0, W1, b1, W2, b2):
    feats = [features_0, features_1, features_2]
    Ws = [W0, W1, W2]
    bs = [b0, b1, b2]
    tiles = [4096, 1024, 256]

    out = None
    for i in range(3):
        B, C, H, Wd = feats[i].shape
        feat = feats[i].reshape(B, C, H * Wd)
        w3 = Ws[i].reshape(_NA, _NO, C)
        b2 = bs[i].reshape(_NA, _NO)
        out = _scale_call(feat, w3, b2, out, tiles[i], _ROW_OFF[i])
    return out


# speedup vs baseline: 1.7404x; 1.2459x over previous
import functools

import jax
import jax.numpy as jnp
from jax.experimental import pallas as pl
from jax.experimental.pallas import tpu as pltpu

_NA = 3
_NC = 80
_NO = 5 + _NC
_HWS = [64 * 64, 32 * 32, 16 * 16]
_TOTAL_ROWS = _NA * sum(_HWS)
_ROW_OFF = [0, _NA * _HWS[0], _NA * (_HWS[0] + _HWS[1])]


def _body(feat_ref, w_ref, b_ref, out_ref, *, hw):
    x = feat_ref[0].astype(jnp.bfloat16)
    for a in range(_NA):
        w = w_ref[a].astype(jnp.bfloat16)
        y = jax.lax.dot_general(
            x, w, (((0,), (1,)), ((), ())), preferred_element_type=jnp.float32
        )
        out_ref[0, a * hw:(a + 1) * hw, :] = y + b_ref[a][None, :]


def _body_acc(feat_ref, w_ref, b_ref, acc_ref, out_ref, *, hw):
    del acc_ref
    _body(feat_ref, w_ref, b_ref, out_ref, hw=hw)


def _scale_call(feat, w3, b2, acc, row_off):
    B, C, HW = feat.shape
    rows = _NA * HW
    grid = (B,)

    feat_spec = pl.BlockSpec((1, C, HW), lambda b: (b, 0, 0))
    w_spec = pl.BlockSpec((_NA, _NO, C), lambda b: (0, 0, 0))
    b_spec = pl.BlockSpec((_NA, _NO), lambda b: (0, 0))
    out_spec = pl.BlockSpec((1, rows, _NO), lambda b, _o=row_off // rows: (b, _o, 0))
    out_shape = jax.ShapeDtypeStruct((B, _TOTAL_ROWS, _NO), jnp.float32)

    params = pltpu.CompilerParams(dimension_semantics=("arbitrary",))
    if acc is None:
        return pl.pallas_call(
            functools.partial(_body, hw=HW),
            grid=grid,
            in_specs=[feat_spec, w_spec, b_spec],
            out_specs=out_spec,
            out_shape=out_shape,
            compiler_params=params,
        )(feat, w3, b2)
    return pl.pallas_call(
        functools.partial(_body_acc, hw=HW),
        grid=grid,
        in_specs=[feat_spec, w_spec, b_spec,
                  pl.BlockSpec(memory_space=pl.ANY)],
        out_specs=out_spec,
        out_shape=out_shape,
        input_output_aliases={3: 0},
        compiler_params=params,
    )(feat, w3, b2, acc)


def kernel(features_0, features_1, features_2, W0, b0, W1, b1, W2, b2):
    feats = [features_0, features_1, features_2]
    Ws = [W0, W1, W2]
    bs = [b0, b1, b2]

    out = None
    for i in range(3):
        B, C, H, Wd = feats[i].shape
        feat = feats[i].reshape(B, C, H * Wd)
        w3 = Ws[i].reshape(_NA, _NO, C)
        b2 = bs[i].reshape(_NA, _NO)
        out = _scale_call(feat, w3, b2, out, _ROW_OFF[i])
    return out


# speedup vs baseline: 1.8087x; 1.0393x over previous
import functools

import jax
import jax.numpy as jnp
from jax.experimental import pallas as pl
from jax.experimental.pallas import tpu as pltpu

_NA = 3
_NC = 80
_NO = 5 + _NC
_HWS = [64 * 64, 32 * 32, 16 * 16]
_TOTAL_ROWS = _NA * sum(_HWS)
_ROW_OFF = [0, _NA * _HWS[0], _NA * (_HWS[0] + _HWS[1])]


def _body(feat_ref, w_ref, b_ref, out_ref, *, hw, bb):
    for j in range(bb):
        x = feat_ref[j].astype(jnp.bfloat16)
        for a in range(_NA):
            w = w_ref[a].astype(jnp.bfloat16)
            y = jax.lax.dot_general(
                x, w, (((0,), (1,)), ((), ())),
                preferred_element_type=jnp.float32,
            )
            out_ref[j, a * hw:(a + 1) * hw, :] = y + b_ref[a][None, :]


def _body_acc(feat_ref, w_ref, b_ref, acc_ref, out_ref, *, hw, bb):
    del acc_ref
    _body(feat_ref, w_ref, b_ref, out_ref, hw=hw, bb=bb)


def _scale_call(feat, w3, b2, acc, row_off, bb):
    B, C, HW = feat.shape
    rows = _NA * HW
    grid = (B // bb,)

    feat_spec = pl.BlockSpec((bb, C, HW), lambda b: (b, 0, 0))
    w_spec = pl.BlockSpec((_NA, _NO, C), lambda b: (0, 0, 0))
    b_spec = pl.BlockSpec((_NA, _NO), lambda b: (0, 0))
    out_spec = pl.BlockSpec((bb, rows, _NO), lambda b, _o=row_off // rows: (b, _o, 0))
    out_shape = jax.ShapeDtypeStruct((B, _TOTAL_ROWS, _NO), jnp.float32)

    params = pltpu.CompilerParams(
        dimension_semantics=("arbitrary",),
        vmem_limit_bytes=100 * 1024 * 1024,
    )
    if acc is None:
        return pl.pallas_call(
            functools.partial(_body, hw=HW, bb=bb),
            grid=grid,
            in_specs=[feat_spec, w_spec, b_spec],
            out_specs=out_spec,
            out_shape=out_shape,
            compiler_params=params,
        )(feat, w3, b2)
    return pl.pallas_call(
        functools.partial(_body_acc, hw=HW, bb=bb),
        grid=grid,
        in_specs=[feat_spec, w_spec, b_spec,
                  pl.BlockSpec(memory_space=pl.ANY)],
        out_specs=out_spec,
        out_shape=out_shape,
        input_output_aliases={3: 0},
        compiler_params=params,
    )(feat, w3, b2, acc)


def kernel(features_0, features_1, features_2, W0, b0, W1, b1, W2, b2):
    feats = [features_0, features_1, features_2]
    Ws = [W0, W1, W2]
    bs = [b0, b1, b2]

    bbs = [2, 4, 8]
    out = None
    for i in range(3):
        B, C, H, Wd = feats[i].shape
        feat = feats[i].reshape(B, C, H * Wd)
        w3 = Ws[i].reshape(_NA, _NO, C)
        b2 = bs[i].reshape(_NA, _NO)
        out = _scale_call(feat, w3, b2, out, _ROW_OFF[i], bbs[i])
    return out
